# baseline (speedup 1.0000x reference)
import functools

import jax
import jax.numpy as jnp
from jax import lax
from jax.experimental import pallas as pl
from jax.experimental.pallas import tpu as pltpu
from jax.experimental.pallas import tpu_sc as plsc

H = 32
W = 32
D = 128
L = 16


def _pos_embed_body(row_hbm, col_hbm, out_hbm, colbuf, rowbuf, outbuf):
    wid = lax.axis_index("s") * 2 + lax.axis_index("c")
    pltpu.sync_copy(col_hbm, colbuf)
    pltpu.sync_copy(row_hbm.at[pl.ds(wid, 1)], rowbuf)
    for k in range(D // L):
        rvec = rowbuf[0, pl.ds(k * L, L)]
        for w in range(W):
            outbuf[w, pl.ds(k * L, L)] = colbuf[w, pl.ds(k * L, L)]
            outbuf[w, pl.ds(D + k * L, L)] = rvec
    pltpu.sync_copy(outbuf, out_hbm.at[pl.ds(wid * W, W)])


@jax.jit
def _pos_embed(row_embed, col_embed):
    mesh = plsc.VectorSubcoreMesh(core_axis_name="c", subcore_axis_name="s")
    kfn = functools.partial(
        pl.kernel,
        mesh=mesh,
        out_type=jax.ShapeDtypeStruct((H * W, 2 * D), jnp.float32),
        scratch_types=[
            pltpu.VMEM((W, D), jnp.float32),
            pltpu.VMEM((1, D), jnp.float32),
            pltpu.VMEM((W, 2 * D), jnp.float32),
        ],
    )(_pos_embed_body)
    return kfn(row_embed, col_embed)


def kernel(tensor, row_embed, col_embed):
    del tensor
    return _pos_embed(row_embed, col_embed)


# speedup vs baseline: 1.0687x; 1.0687x over previous
import functools

import jax
import jax.numpy as jnp
from jax import lax
from jax.experimental import pallas as pl
from jax.experimental.pallas import tpu as pltpu
from jax.experimental.pallas import tpu_sc as plsc

H = 32
W = 32
D = 128
L = 16


def _pos_embed_body(row_hbm, col_hbm, out_hbm, rowbuf, outbuf, sem_c, sem_r):
    wid = lax.axis_index("s") * 2 + lax.axis_index("c")
    cp_c = pltpu.make_async_copy(col_hbm, outbuf.at[:, pl.ds(0, D)], sem_c)
    cp_r = pltpu.make_async_copy(row_hbm.at[pl.ds(wid, 1)], rowbuf, sem_r)
    cp_c.start()
    cp_r.start()
    cp_r.wait()
    rvecs = [rowbuf[0, pl.ds(k * L, L)] for k in range(D // L)]

    def bcast(w, carry):
        for k in range(D // L):
            outbuf[w, pl.ds(D + k * L, L)] = rvecs[k]
        return carry

    lax.fori_loop(0, W, bcast, 0, unroll=False)
    cp_c.wait()
    pltpu.sync_copy(outbuf, out_hbm.at[pl.ds(wid * W, W)])


@jax.jit
def _pos_embed(row_embed, col_embed):
    mesh = plsc.VectorSubcoreMesh(core_axis_name="c", subcore_axis_name="s")
    kfn = functools.partial(
        pl.kernel,
        mesh=mesh,
        out_type=jax.ShapeDtypeStruct((H * W, 2 * D), jnp.float32),
        scratch_types=[
            pltpu.VMEM((1, D), jnp.float32),
            pltpu.VMEM((W, 2 * D), jnp.float32),
            pltpu.SemaphoreType.DMA,
            pltpu.SemaphoreType.DMA,
        ],
    )(_pos_embed_body)
    return kfn(row_embed, col_embed)


def kernel(tensor, row_embed, col_embed):
    del tensor
    return _pos_embed(row_embed, col_embed)


# speedup vs baseline: 1.1321x; 1.0593x over previous
import functools

import jax
import jax.numpy as jnp
from jax import lax
from jax.experimental import pallas as pl
from jax.experimental.pallas import tpu as pltpu
from jax.experimental.pallas import tpu_sc as plsc

H = 32
W = 32
D = 128
L = 16
NCORES = 1
NSUB = 16
NWORK = NCORES * NSUB
HPW = H // NWORK
RPW = HPW * W


def _pos_embed_body(row_hbm, col_hbm, out_hbm, rowbuf, outbuf, sem_c, sem_r):
    wid = lax.axis_index("s") * NCORES + lax.axis_index("c")
    copies = [
        pltpu.make_async_copy(
            col_hbm, outbuf.at[pl.ds(j * W, W), pl.ds(0, D)], sem_c)
        for j in range(HPW)
    ]
    cp_r = pltpu.make_async_copy(
        row_hbm.at[pl.ds(wid * HPW, HPW)], rowbuf, sem_r)
    for cp in copies:
        cp.start()
    cp_r.start()
    cp_r.wait()
    rvecs = [[rowbuf[j, pl.ds(k * L, L)] for k in range(D // L)]
             for j in range(HPW)]

    def bcast(w, carry):
        for j in range(HPW):
            for k in range(D // L):
                outbuf[j * W + w, pl.ds(D + k * L, L)] = rvecs[j][k]
        return carry

    lax.fori_loop(0, W, bcast, 0, unroll=False)
    for cp in copies:
        cp.wait()
    pltpu.sync_copy(outbuf, out_hbm.at[pl.ds(wid * RPW, RPW)])


@jax.jit
def _pos_embed(row_embed, col_embed):
    mesh = plsc.VectorSubcoreMesh(
        core_axis_name="c", subcore_axis_name="s",
        num_cores=NCORES, num_subcores=NSUB)
    kfn = functools.partial(
        pl.kernel,
        mesh=mesh,
        out_type=jax.ShapeDtypeStruct((H * W, 2 * D), jnp.float32),
        scratch_types=[
            pltpu.VMEM((HPW, D), jnp.float32),
            pltpu.VMEM((RPW, 2 * D), jnp.float32),
            pltpu.SemaphoreType.DMA,
            pltpu.SemaphoreType.DMA,
        ],
    )(_pos_embed_body)
    return kfn(row_embed, col_embed)


def kernel(tensor, row_embed, col_embed):
    del tensor
    return _pos_embed(row_embed, col_embed)
